# Initial kernel scaffold; baseline (speedup 1.0000x reference)
import functools

import jax
import jax.numpy as jnp
from jax import lax
from jax.experimental import pallas as pl
from jax.experimental.pallas import tpu as pltpu
from jax.experimental.pallas import tpu_sc as plsc

N = 10000
D = 256
F = 256
S = 3
E = 160000

NC = 2
NS = 16
FH = F // NC
CHUNK = 128

E_TOT = S * E
E_PER_TILE = -(-E_TOT // (NS * CHUNK)) * CHUNK
E_PAD = E_PER_TILE * NS
N_CHUNKS = E_PER_TILE // CHUNK

ROWS_PER_TILE = N // NS
COPY_BLK = 125
N_COPY = ROWS_PER_TILE // COPY_BLK

NB = 1000


def _mm_body(x_ref, w_ref, o_ref):
    o_ref[...] = jnp.dot(x_ref[...], w_ref[0], preferred_element_type=jnp.float32)


def _make_table(x, kernels):
    grid = (N // NB, NC, S)
    return pl.pallas_call(
        _mm_body,
        grid=grid,
        in_specs=[
            pl.BlockSpec((NB, D), lambda nb, c, i: (nb, 0)),
            pl.BlockSpec((1, D, FH), lambda nb, c, i: (i, 0, c)),
        ],
        out_specs=pl.BlockSpec(
            (NB, FH), lambda nb, c, i: (c * (S * N // NB) + i * (N // NB) + nb, 0)
        ),
        out_shape=jax.ShapeDtypeStruct((NC * S * N, FH), jnp.float32),
    )(x, kernels)


def _sc_body(table, cols, rows, vals, bias, out,
             cidx_v, ridx_v, vals_v, gbuf, bias_v, acc, sem):
    cid = lax.axis_index("c")
    sid = lax.axis_index("s")

    pltpu.sync_copy(bias.at[pl.ds(cid * FH, FH)], bias_v)
    bvecs = [bias_v[pl.ds(k * 16, 16)] for k in range(FH // 16)]

    def fill_row(j, carry):
        for k in range(FH // 16):
            gbuf[j, pl.ds(k * 16, 16)] = bvecs[k]
        return carry

    lax.fori_loop(0, COPY_BLK, fill_row, 0)
    base = sid * ROWS_PER_TILE
    for t in range(N_COPY):
        pltpu.sync_copy(gbuf.at[pl.ds(0, COPY_BLK)],
                        acc.at[pl.ds(base + t * COPY_BLK, COPY_BLK)])
    plsc.subcore_barrier()

    ebase = sid * E_PER_TILE
    tab_off = cid * (S * N)

    def chunk_body(t, carry):
        b = ebase + t * CHUNK
        pltpu.sync_copy(cols.at[pl.ds(b, CHUNK)], cidx_v)
        pltpu.sync_copy(rows.at[pl.ds(b, CHUNK)], ridx_v)
        pltpu.sync_copy(vals.at[pl.ds(b, CHUNK)], vals_v)
        for k in range(CHUNK // 16):
            cidx_v[pl.ds(k * 16, 16)] = cidx_v[pl.ds(k * 16, 16)] + tab_off
        pltpu.async_copy(table.at[cidx_v], gbuf, sem).wait()

        def scale(j, c2):
            vj = plsc.load_gather(vals_v, [jnp.zeros((16,), jnp.int32) + j])
            for k in range(FH // 16):
                gbuf[j, pl.ds(k * 16, 16)] = gbuf[j, pl.ds(k * 16, 16)] * vj
            return c2

        lax.fori_loop(0, CHUNK, scale, 0)
        pltpu.sync_copy(gbuf, acc.at[ridx_v], add=True)
        return carry

    lax.fori_loop(0, N_CHUNKS, chunk_body, 0)
    plsc.subcore_barrier()

    for t in range(N_COPY):
        r0 = base + t * COPY_BLK
        pltpu.sync_copy(acc.at[pl.ds(r0, COPY_BLK)], gbuf.at[pl.ds(0, COPY_BLK)])
        pltpu.sync_copy(gbuf.at[pl.ds(0, COPY_BLK)],
                        out.at[pl.ds(r0, COPY_BLK), pl.ds(cid * FH, FH)])


_sc_call = functools.partial(
    pl.kernel,
    out_type=jax.ShapeDtypeStruct((N, F), jnp.float32),
    mesh=plsc.VectorSubcoreMesh(core_axis_name="c", subcore_axis_name="s"),
    scratch_types=[
        pltpu.VMEM((CHUNK,), jnp.int32),
        pltpu.VMEM((CHUNK,), jnp.int32),
        pltpu.VMEM((CHUNK,), jnp.float32),
        pltpu.VMEM((CHUNK, FH), jnp.float32),
        pltpu.VMEM((FH,), jnp.float32),
        pltpu.VMEM_SHARED((N, FH), jnp.float32),
        pltpu.SemaphoreType.DMA,
    ],
)(_sc_body)


@jax.jit
def kernel(inputs, kernels, bias, sup_vals, sup_rows, sup_cols):
    x = inputs[0]
    table = _make_table(x, kernels)

    off = (jnp.arange(S, dtype=jnp.int32) * N)[:, None]
    cols = (sup_cols + off).reshape(-1)
    rows = sup_rows.reshape(-1)
    vals = sup_vals.reshape(-1)
    pad = E_PAD - E_TOT
    cols = jnp.concatenate([cols, jnp.zeros((pad,), jnp.int32)])
    rows = jnp.concatenate([rows, jnp.zeros((pad,), jnp.int32)])
    vals = jnp.concatenate([vals, jnp.zeros((pad,), jnp.float32)])

    out = _sc_call(table, cols, rows, vals, bias)
    return out[None]


# speedup vs baseline: 1.8883x; 1.8883x over previous
import functools

import jax
import jax.numpy as jnp
from jax import lax
from jax.experimental import pallas as pl
from jax.experimental.pallas import tpu as pltpu
from jax.experimental.pallas import tpu_sc as plsc

N = 10000
D = 256
F = 256
S = 3
E = 160000

NC = 2
NS = 16
FH = F // NC
CHUNK = 128

E_TOT = S * E
E_PER_TILE = -(-E_TOT // (NS * CHUNK)) * CHUNK
E_PAD = E_PER_TILE * NS
N_CHUNKS = E_PER_TILE // CHUNK

OUT_N = 10240
ROWS_PER_TILE = OUT_N // NS
COPY_BLK = 128
N_COPY = ROWS_PER_TILE // COPY_BLK

NB = 1000


def _mm_body(x_ref, w_ref, o_ref):
    o_ref[...] = jnp.dot(x_ref[...], w_ref[0], preferred_element_type=jnp.float32)


def _make_table(x, kernels):
    grid = (N // NB, NC, S)
    return pl.pallas_call(
        _mm_body,
        grid=grid,
        in_specs=[
            pl.BlockSpec((NB, D), lambda nb, c, i: (nb, 0)),
            pl.BlockSpec((1, D, FH), lambda nb, c, i: (i, 0, c)),
        ],
        out_specs=pl.BlockSpec(
            (NB, FH), lambda nb, c, i: (c * (S * N // NB) + i * (N // NB) + nb, 0)
        ),
        out_shape=jax.ShapeDtypeStruct((NC * S * N, FH), jnp.float32),
    )(x, kernels)


def _sc_body(table, cols, rows, vals, bias, out,
             cidx_v, ridx_v, vals_v, gbuf, bias_v, acc, sem):
    cid = lax.axis_index("c")
    sid = lax.axis_index("s")

    pltpu.sync_copy(bias.at[pl.ds(cid * FH, FH)], bias_v)
    bvecs = [bias_v[pl.ds(k * 16, 16)] for k in range(FH // 16)]

    def fill_row(j, carry):
        for k in range(FH // 16):
            gbuf[j, pl.ds(k * 16, 16)] = bvecs[k]
        return carry

    lax.fori_loop(0, COPY_BLK, fill_row, 0)
    base = sid * ROWS_PER_TILE
    for t in range(N_COPY):
        pltpu.sync_copy(gbuf.at[pl.ds(0, COPY_BLK)],
                        acc.at[pl.ds(base + t * COPY_BLK, COPY_BLK)])
    plsc.subcore_barrier()

    ebase = sid * E_PER_TILE
    tab_off = cid * (S * N)

    def chunk_body(t, carry):
        b = ebase + t * CHUNK
        pltpu.sync_copy(cols.at[pl.ds(b, CHUNK)], cidx_v)
        pltpu.sync_copy(rows.at[pl.ds(b, CHUNK)], ridx_v)
        pltpu.sync_copy(vals.at[pl.ds(b * 16, CHUNK * 16)], vals_v)
        for k in range(CHUNK // 16):
            cidx_v[pl.ds(k * 16, 16)] = cidx_v[pl.ds(k * 16, 16)] + tab_off
        pltpu.async_copy(table.at[cidx_v], gbuf, sem).wait()

        def scale(j, c2):
            vj = vals_v[pl.ds(j * 16, 16)]
            for k in range(FH // 16):
                gbuf[j, pl.ds(k * 16, 16)] = gbuf[j, pl.ds(k * 16, 16)] * vj
            return c2

        lax.fori_loop(0, CHUNK, scale, 0)
        pltpu.sync_copy(gbuf, acc.at[ridx_v], add=True)
        return carry

    lax.fori_loop(0, N_CHUNKS, chunk_body, 0)
    plsc.subcore_barrier()

    for t in range(N_COPY):
        r0 = base + t * COPY_BLK
        pltpu.sync_copy(acc.at[pl.ds(r0, COPY_BLK)], gbuf.at[pl.ds(0, COPY_BLK)])
        pltpu.sync_copy(gbuf.at[pl.ds(0, COPY_BLK)],
                        out.at[pl.ds(r0, COPY_BLK), pl.ds(cid * FH, FH)])


@functools.cache
def _sc_call():
    return functools.partial(
        pl.kernel,
        out_type=jax.ShapeDtypeStruct((OUT_N, F), jnp.float32),
        mesh=plsc.VectorSubcoreMesh(core_axis_name="c", subcore_axis_name="s"),
        scratch_types=[
            pltpu.VMEM((CHUNK,), jnp.int32),
            pltpu.VMEM((CHUNK,), jnp.int32),
            pltpu.VMEM((CHUNK * 16,), jnp.float32),
            pltpu.VMEM((CHUNK, FH), jnp.float32),
            pltpu.VMEM((FH,), jnp.float32),
            pltpu.VMEM_SHARED((OUT_N, FH), jnp.float32),
            pltpu.SemaphoreType.DMA,
        ],
    )(_sc_body)


@jax.jit
def kernel(inputs, kernels, bias, sup_vals, sup_rows, sup_cols):
    x = inputs[0]
    table = _make_table(x, kernels)

    off = (jnp.arange(S, dtype=jnp.int32) * N)[:, None]
    cols = (sup_cols + off).reshape(-1)
    rows = sup_rows.reshape(-1)
    vals = sup_vals.reshape(-1)
    pad = E_PAD - E_TOT
    cols = jnp.concatenate([cols, jnp.zeros((pad,), jnp.int32)])
    rows = jnp.concatenate([rows, jnp.zeros((pad,), jnp.int32)])
    vals = jnp.concatenate([vals, jnp.zeros((pad,), jnp.float32)])
    vals = jnp.broadcast_to(vals[:, None], (E_PAD, 16)).reshape(-1)

    out = _sc_call()(table, cols, rows, vals, bias)
    return out[None, :N]
